# Initial kernel scaffold; baseline (speedup 1.0000x reference)
import functools

import jax
import jax.numpy as jnp
from jax import lax
from jax.experimental import pallas as pl
from jax.experimental.pallas import tpu as pltpu
from jax.experimental.pallas import tpu_sc as plsc

N_NODES = 10000
D = 128
C = 40
NP = 10240
E = 320000
NC, NS = 2, 16
NWORK = NC * NS
K = 128
EP = 327680
NWIN = EP // (NWORK * K)
RPT = NP // NS
BLK = 1024

_mesh = plsc.VectorSubcoreMesh(core_axis_name="c", subcore_axis_name="s")


@functools.partial(
    pl.kernel,
    out_type=jax.ShapeDtypeStruct((NC, NP), jnp.float32),
    mesh=_mesh,
    scratch_types=[
        pltpu.VMEM((K,), jnp.int32),
        pltpu.VMEM((K,), jnp.float32),
        pltpu.VMEM((RPT,), jnp.float32),
        pltpu.VMEM_SHARED((NP,), jnp.float32),
    ],
)
def _deg_kernel(dst_hbm, deg_out, idx_v, ones_v, zslab_v, acc):
    c = lax.axis_index("c")
    s = lax.axis_index("s")

    def initz(i, _):
        zslab_v[pl.ds(16 * i, 16)] = jnp.zeros((16,), jnp.float32)
        return 0

    lax.fori_loop(0, RPT // 16, initz, 0)

    def init1(i, _):
        ones_v[pl.ds(16 * i, 16)] = jnp.ones((16,), jnp.float32)
        return 0

    lax.fori_loop(0, K // 16, init1, 0)

    pltpu.sync_copy(zslab_v, acc.at[pl.ds(s * RPT, RPT)])
    plsc.subcore_barrier()

    def step(w, _):
        pltpu.sync_copy(dst_hbm.at[c, s, w], idx_v)
        pltpu.sync_copy(ones_v, acc.at[idx_v], add=True)
        return 0

    lax.fori_loop(0, NWIN, step, 0)
    plsc.subcore_barrier()
    pltpu.sync_copy(acc.at[pl.ds(s * RPT, RPT)],
                    deg_out.at[c, pl.ds(s * RPT, RPT)])


@functools.partial(
    pl.kernel,
    out_type=jax.ShapeDtypeStruct((NC, NP, D), jnp.float32),
    mesh=_mesh,
    scratch_types=[
        pltpu.VMEM((K,), jnp.int32),
        pltpu.VMEM((K,), jnp.int32),
        pltpu.VMEM((K, D), jnp.float32),
        pltpu.VMEM_SHARED((NP, D), jnp.float32),
        pltpu.SemaphoreType.DMA,
    ],
)
def _agg_kernel(v_hbm, src_hbm, dst_hbm, s_out, sidx, didx, rows, acc, sem):
    c = lax.axis_index("c")
    s = lax.axis_index("s")

    def zrows(i, _):
        for j in range(D // 16):
            rows[i, pl.ds(16 * j, 16)] = jnp.zeros((16,), jnp.float32)
        return 0

    lax.fori_loop(0, K, zrows, 0)
    for k in range(RPT // K):
        pltpu.sync_copy(rows, acc.at[pl.ds(s * RPT + k * K, K)])
    plsc.subcore_barrier()

    def step(w, _):
        pltpu.sync_copy(src_hbm.at[c, s, w], sidx)
        pltpu.sync_copy(dst_hbm.at[c, s, w], didx)
        pltpu.async_copy(v_hbm.at[sidx], rows, sem).wait()
        pltpu.sync_copy(rows, acc.at[didx], add=True)
        return 0

    lax.fori_loop(0, NWIN, step, 0)
    plsc.subcore_barrier()
    pltpu.sync_copy(acc.at[pl.ds(s * RPT, RPT)],
                    s_out.at[c, pl.ds(s * RPT, RPT)])


def _prep_body(d0, d1, x, dinv_o, v0_o):
    i = pl.program_id(0)
    deg = d0[...] + d1[...] + 1.0
    dinv = lax.rsqrt(deg)
    row = i * BLK + lax.broadcasted_iota(jnp.int32, (BLK, 1), 0)
    dinv = jnp.where(row < N_NODES, dinv, 0.0)
    dinv_o[...] = dinv
    v0_o[...] = x[...] * dinv


_prep_call = pl.pallas_call(
    _prep_body,
    grid=(NP // BLK,),
    in_specs=[
        pl.BlockSpec((BLK, 1), lambda i: (i, 0)),
        pl.BlockSpec((BLK, 1), lambda i: (i, 0)),
        pl.BlockSpec((BLK, D), lambda i: (i, 0)),
    ],
    out_specs=[
        pl.BlockSpec((BLK, 1), lambda i: (i, 0)),
        pl.BlockSpec((BLK, D), lambda i: (i, 0)),
    ],
    out_shape=[
        jax.ShapeDtypeStruct((NP, 1), jnp.float32),
        jax.ShapeDtypeStruct((NP, D), jnp.float32),
    ],
)


def _layer_body(s0, s1, v, dinv, w, b, vo):
    t = (s0[...] + s1[...] + v[...]) * dinv[...]
    h = jnp.dot(t, w[...], preferred_element_type=jnp.float32) + b[...]
    vo[...] = jnp.maximum(h, 0.0) * dinv[...]


_layer_call = pl.pallas_call(
    _layer_body,
    grid=(NP // BLK,),
    in_specs=[
        pl.BlockSpec((BLK, D), lambda i: (i, 0)),
        pl.BlockSpec((BLK, D), lambda i: (i, 0)),
        pl.BlockSpec((BLK, D), lambda i: (i, 0)),
        pl.BlockSpec((BLK, 1), lambda i: (i, 0)),
        pl.BlockSpec((D, D), lambda i: (0, 0)),
        pl.BlockSpec((1, D), lambda i: (0, 0)),
    ],
    out_specs=pl.BlockSpec((BLK, D), lambda i: (i, 0)),
    out_shape=jax.ShapeDtypeStruct((NP, D), jnp.float32),
)


def _final_body(s0, s1, v, dinv, w, b, out, accs):
    i = pl.program_id(0)

    @pl.when(i == 0)
    def _():
        accs[...] = jnp.zeros_like(accs)

    t = (s0[...] + s1[...] + v[...]) * dinv[...]
    h = jnp.dot(t, w[...], preferred_element_type=jnp.float32) + b[...]
    row = i * BLK + lax.broadcasted_iota(jnp.int32, (BLK, 1), 0)
    h = jnp.where(row < N_NODES, h, 0.0)
    accs[...] += jnp.sum(h, axis=0, keepdims=True)

    @pl.when(i == pl.num_programs(0) - 1)
    def _():
        pooled = accs[...]
        lane = lax.broadcasted_iota(jnp.int32, (1, 128), 1)
        valid = lane < C
        m = jnp.max(jnp.where(valid, pooled, jnp.float32(-1e30)),
                    axis=1, keepdims=True)
        e = jnp.where(valid, jnp.exp(pooled - m), 0.0)
        ls = pooled - (m + jnp.log(jnp.sum(e, axis=1, keepdims=True)))
        out[0:1, :] = pooled
        out[1:2, :] = ls


_final_call = pl.pallas_call(
    _final_body,
    grid=(NP // BLK,),
    in_specs=[
        pl.BlockSpec((BLK, D), lambda i: (i, 0)),
        pl.BlockSpec((BLK, D), lambda i: (i, 0)),
        pl.BlockSpec((BLK, D), lambda i: (i, 0)),
        pl.BlockSpec((BLK, 1), lambda i: (i, 0)),
        pl.BlockSpec((D, D), lambda i: (0, 0)),
        pl.BlockSpec((1, D), lambda i: (0, 0)),
    ],
    out_specs=pl.BlockSpec((2, 128), lambda i: (0, 0)),
    out_shape=jax.ShapeDtypeStruct((2, 128), jnp.float32),
    scratch_shapes=[pltpu.VMEM((1, 128), jnp.float32)],
)


def kernel(x, edge_index, W1, b1, W2, b2, W3, b3, W4, b4):
    src = edge_index[0]
    dst = edge_index[1]
    pad_idx = N_NODES + (jnp.arange(EP - E, dtype=jnp.int32) % (NP - N_NODES))
    srcp = jnp.concatenate([src, pad_idx]).reshape(NC, NS, NWIN, K)
    dstp = jnp.concatenate([dst, pad_idx]).reshape(NC, NS, NWIN, K)
    xp = jnp.pad(x, ((0, NP - N_NODES), (0, 0)))

    degp = _deg_kernel(dstp)
    dinv, v = _prep_call(degp[0].reshape(NP, 1), degp[1].reshape(NP, 1), xp)

    for (W, b) in ((W1, b1), (W2, b2), (W3, b3)):
        sp = _agg_kernel(v, srcp, dstp)
        v = _layer_call(sp[0], sp[1], v, dinv, W, b.reshape(1, D))

    sp = _agg_kernel(v, srcp, dstp)
    W4p = jnp.pad(W4, ((0, 0), (0, 128 - C)))
    b4p = jnp.pad(b4, (0, 128 - C)).reshape(1, 128)
    out = _final_call(sp[0], sp[1], v, dinv, W4p, b4p)
    return (out[0:1, :C], out[1:2, :C])


# speedup vs baseline: 12.2024x; 12.2024x over previous
import functools

import jax
import jax.numpy as jnp
from jax import lax
from jax.experimental import pallas as pl
from jax.experimental.pallas import tpu as pltpu
from jax.experimental.pallas import tpu_sc as plsc

N_NODES = 10000
D = 128
C = 40
NP = 10240
E = 320000
NC, NS = 2, 16
NWORK = NC * NS
K = 128
EP = 327680
NWIN = EP // (NWORK * K)
RPT = NP // NS
BLK = 1024


def _deg_body(dst_hbm, deg_out, idx_v, ones_v, zslab_v, acc):
    c = lax.axis_index("c")
    s = lax.axis_index("s")

    def initz(i, _):
        zslab_v[pl.ds(16 * i, 16)] = jnp.zeros((16,), jnp.float32)
        return 0

    lax.fori_loop(0, RPT // 16, initz, 0)

    def init1(i, _):
        ones_v[pl.ds(16 * i, 16)] = jnp.ones((16,), jnp.float32)
        return 0

    lax.fori_loop(0, K // 16, init1, 0)

    pltpu.sync_copy(zslab_v, acc.at[pl.ds(s * RPT, RPT)])
    plsc.subcore_barrier()

    def step(w, _):
        pltpu.sync_copy(dst_hbm.at[c, s, w], idx_v)
        pltpu.sync_copy(ones_v, acc.at[idx_v], add=True)
        return 0

    lax.fori_loop(0, NWIN, step, 0)
    plsc.subcore_barrier()
    pltpu.sync_copy(acc.at[pl.ds(s * RPT, RPT)],
                    deg_out.at[c, pl.ds(s * RPT, RPT)])


def _agg_body(v_hbm, src_hbm, dst_hbm, s_out, sidx, didx, rows, acc, sem):
    c = lax.axis_index("c")
    s = lax.axis_index("s")

    def zrows(i, _):
        for j in range(D // 16):
            rows[i, pl.ds(16 * j, 16)] = jnp.zeros((16,), jnp.float32)
        return 0

    lax.fori_loop(0, K, zrows, 0)
    for k in range(RPT // K):
        pltpu.sync_copy(rows, acc.at[pl.ds(s * RPT + k * K, K)])
    plsc.subcore_barrier()

    def step(w, _):
        pltpu.sync_copy(src_hbm.at[c, s, w], sidx)
        pltpu.sync_copy(dst_hbm.at[c, s, w], didx)
        pltpu.async_copy(v_hbm.at[sidx], rows, sem).wait()
        pltpu.sync_copy(rows, acc.at[didx], add=True)
        return 0

    lax.fori_loop(0, NWIN, step, 0)
    plsc.subcore_barrier()
    pltpu.sync_copy(acc.at[pl.ds(s * RPT, RPT)],
                    s_out.at[c, pl.ds(s * RPT, RPT)])


@functools.cache
def _sc_kernels():
    mesh = plsc.VectorSubcoreMesh(core_axis_name="c", subcore_axis_name="s")
    deg = pl.kernel(
        _deg_body,
        out_type=jax.ShapeDtypeStruct((NC, NP), jnp.float32),
        mesh=mesh,
        scratch_types=[
            pltpu.VMEM((K,), jnp.int32),
            pltpu.VMEM((K,), jnp.float32),
            pltpu.VMEM((RPT,), jnp.float32),
            pltpu.VMEM_SHARED((NP,), jnp.float32),
        ],
    )
    agg = pl.kernel(
        _agg_body,
        out_type=jax.ShapeDtypeStruct((NC, NP, D), jnp.float32),
        mesh=mesh,
        scratch_types=[
            pltpu.VMEM((K,), jnp.int32),
            pltpu.VMEM((K,), jnp.int32),
            pltpu.VMEM((K, D), jnp.float32),
            pltpu.VMEM_SHARED((NP, D), jnp.float32),
            pltpu.SemaphoreType.DMA,
        ],
    )
    return deg, agg


def _prep_body(d0, d1, x, dinv_o, v0_o):
    i = pl.program_id(0)
    deg = d0[...] + d1[...] + 1.0
    dinv = lax.rsqrt(deg)
    row = i * BLK + lax.broadcasted_iota(jnp.int32, (BLK, 1), 0)
    dinv = jnp.where(row < N_NODES, dinv, 0.0)
    dinv_o[...] = dinv
    v0_o[...] = x[...] * dinv


_prep_call = pl.pallas_call(
    _prep_body,
    grid=(NP // BLK,),
    in_specs=[
        pl.BlockSpec((BLK, 1), lambda i: (i, 0)),
        pl.BlockSpec((BLK, 1), lambda i: (i, 0)),
        pl.BlockSpec((BLK, D), lambda i: (i, 0)),
    ],
    out_specs=[
        pl.BlockSpec((BLK, 1), lambda i: (i, 0)),
        pl.BlockSpec((BLK, D), lambda i: (i, 0)),
    ],
    out_shape=[
        jax.ShapeDtypeStruct((NP, 1), jnp.float32),
        jax.ShapeDtypeStruct((NP, D), jnp.float32),
    ],
)


def _layer_body(s0, s1, v, dinv, w, b, vo):
    t = (s0[...] + s1[...] + v[...]) * dinv[...]
    h = jnp.dot(t, w[...], preferred_element_type=jnp.float32) + b[...]
    vo[...] = jnp.maximum(h, 0.0) * dinv[...]


_layer_call = pl.pallas_call(
    _layer_body,
    grid=(NP // BLK,),
    in_specs=[
        pl.BlockSpec((BLK, D), lambda i: (i, 0)),
        pl.BlockSpec((BLK, D), lambda i: (i, 0)),
        pl.BlockSpec((BLK, D), lambda i: (i, 0)),
        pl.BlockSpec((BLK, 1), lambda i: (i, 0)),
        pl.BlockSpec((D, D), lambda i: (0, 0)),
        pl.BlockSpec((1, D), lambda i: (0, 0)),
    ],
    out_specs=pl.BlockSpec((BLK, D), lambda i: (i, 0)),
    out_shape=jax.ShapeDtypeStruct((NP, D), jnp.float32),
)


def _final_body(s0, s1, v, dinv, w, b, out, accs):
    i = pl.program_id(0)

    @pl.when(i == 0)
    def _():
        accs[...] = jnp.zeros_like(accs)

    t = (s0[...] + s1[...] + v[...]) * dinv[...]
    h = jnp.dot(t, w[...], preferred_element_type=jnp.float32) + b[...]
    row = i * BLK + lax.broadcasted_iota(jnp.int32, (BLK, 1), 0)
    h = jnp.where(row < N_NODES, h, 0.0)
    accs[...] += jnp.sum(h, axis=0, keepdims=True)

    @pl.when(i == pl.num_programs(0) - 1)
    def _():
        pooled = accs[...]
        lane = lax.broadcasted_iota(jnp.int32, (1, 128), 1)
        valid = lane < C
        m = jnp.max(jnp.where(valid, pooled, jnp.float32(-1e30)),
                    axis=1, keepdims=True)
        e = jnp.where(valid, jnp.exp(pooled - m), 0.0)
        ls = pooled - (m + jnp.log(jnp.sum(e, axis=1, keepdims=True)))
        out[0:1, :] = pooled
        out[1:2, :] = ls


_final_call = pl.pallas_call(
    _final_body,
    grid=(NP // BLK,),
    in_specs=[
        pl.BlockSpec((BLK, D), lambda i: (i, 0)),
        pl.BlockSpec((BLK, D), lambda i: (i, 0)),
        pl.BlockSpec((BLK, D), lambda i: (i, 0)),
        pl.BlockSpec((BLK, 1), lambda i: (i, 0)),
        pl.BlockSpec((D, D), lambda i: (0, 0)),
        pl.BlockSpec((1, D), lambda i: (0, 0)),
    ],
    out_specs=pl.BlockSpec((2, 128), lambda i: (0, 0)),
    out_shape=jax.ShapeDtypeStruct((2, 128), jnp.float32),
    scratch_shapes=[pltpu.VMEM((1, 128), jnp.float32)],
)


def kernel(x, edge_index, W1, b1, W2, b2, W3, b3, W4, b4):
    src = edge_index[0]
    dst = edge_index[1]
    pad_idx = N_NODES + (jnp.arange(EP - E, dtype=jnp.int32) % (NP - N_NODES))
    srcp = jnp.concatenate([src, pad_idx]).reshape(NC, NS, NWIN, K)
    dstp = jnp.concatenate([dst, pad_idx]).reshape(NC, NS, NWIN, K)
    xp = jnp.pad(x, ((0, NP - N_NODES), (0, 0)))

    deg_kernel, agg_kernel = _sc_kernels()
    degp = deg_kernel(dstp)
    dinv, v = _prep_call(degp[0].reshape(NP, 1), degp[1].reshape(NP, 1), xp)

    for (W, b) in ((W1, b1), (W2, b2), (W3, b3)):
        sp = agg_kernel(v, srcp, dstp)
        v = _layer_call(sp[0], sp[1], v, dinv, W, b.reshape(1, D))

    sp = agg_kernel(v, srcp, dstp)
    W4p = jnp.pad(W4, ((0, 0), (0, 128 - C)))
    b4p = jnp.pad(b4, (0, 128 - C)).reshape(1, 128)
    out = _final_call(sp[0], sp[1], v, dinv, W4p, b4p)
    return (out[0:1, :C], out[1:2, :C])


# speedup vs baseline: 21.5565x; 1.7666x over previous
import functools

import jax
import jax.numpy as jnp
from jax import lax
from jax.experimental import pallas as pl
from jax.experimental.pallas import tpu as pltpu
from jax.experimental.pallas import tpu_sc as plsc

N_NODES = 10000
D = 128
C = 40
NP = 10240
E = 320000
NC, NS = 2, 16
NWORK = NC * NS
K = 128
EP = 327680
NWIN = EP // (NWORK * K)
RPT = NP // NS
BLK = 1024


def _deg_body(dst_hbm, deg_out, didx_all, ones_v, zslab_v, acc, ssem):
    c = lax.axis_index("c")
    s = lax.axis_index("s")

    def initz(i, _):
        zslab_v[pl.ds(16 * i, 16)] = jnp.zeros((16,), jnp.float32)
        return 0

    lax.fori_loop(0, RPT // 16, initz, 0)

    def init1(i, _):
        ones_v[pl.ds(16 * i, 16)] = jnp.ones((16,), jnp.float32)
        return 0

    lax.fori_loop(0, K // 16, init1, 0)

    pltpu.sync_copy(dst_hbm.at[c, s], didx_all)
    pltpu.sync_copy(zslab_v, acc.at[pl.ds(s * RPT, RPT)])
    plsc.subcore_barrier()

    def fire(w, _):
        pltpu.async_copy(ones_v, acc.at[didx_all.at[w]], ssem, add=True)
        return 0

    lax.fori_loop(0, NWIN, fire, 0)

    def drain(w, _):
        pltpu.make_async_copy(ones_v, acc.at[didx_all.at[w]], ssem).wait()
        return 0

    lax.fori_loop(0, NWIN, drain, 0)
    plsc.subcore_barrier()
    pltpu.sync_copy(acc.at[pl.ds(s * RPT, RPT)],
                    deg_out.at[c, pl.ds(s * RPT, RPT)])


NH = 2
NUH = NWIN // NH


def _agg_body(v_hbm, src_hbm, dst_hbm, s_out, sidx_h, didx_h,
              rows0, rows1, acc, gsem0, gsem1, ssem0, ssem1):
    c = lax.axis_index("c")
    s = lax.axis_index("s")
    bufs = (rows0, rows1)
    gsems = (gsem0, gsem1)
    ssems = (ssem0, ssem1)

    def zrows(i, _):
        for j in range(D // 16):
            rows0[i, pl.ds(16 * j, 16)] = jnp.zeros((16,), jnp.float32)
        return 0

    lax.fori_loop(0, K, zrows, 0)
    base = s * RPT
    for k in range(RPT // K):
        pltpu.sync_copy(rows0, acc.at[pl.ds(base + k * K, K)])
    plsc.subcore_barrier()

    def g_issue(p, u):
        pltpu.async_copy(v_hbm.at[sidx_h.at[u]], bufs[p], gsems[p])

    def g_wait(p, u):
        pltpu.make_async_copy(v_hbm.at[sidx_h.at[u]], bufs[p],
                              gsems[p]).wait()

    def s_issue(p, u):
        pltpu.async_copy(bufs[p], acc.at[didx_h.at[u]], ssems[p], add=True)

    def s_wait(p, u):
        pltpu.make_async_copy(bufs[p], acc.at[didx_h.at[u]], ssems[p]).wait()

    for h in range(NH):
        pltpu.sync_copy(src_hbm.at[c, s, pl.ds(h * NUH, NUH)], sidx_h)
        pltpu.sync_copy(dst_hbm.at[c, s, pl.ds(h * NUH, NUH)], didx_h)

        g_issue(0, 0)
        g_wait(0, 0)
        s_issue(0, 0)
        g_issue(1, 1)

        def step(k, _):
            ua = 2 * k + 1
            g_wait(1, ua)
            s_issue(1, ua)
            s_wait(0, ua - 1)
            g_issue(0, ua + 1)
            ub = 2 * k + 2
            g_wait(0, ub)
            s_issue(0, ub)
            s_wait(1, ub - 1)
            g_issue(1, ub + 1)
            return 0

        lax.fori_loop(0, (NUH - 2) // 2, step, 0)

        u_last = NUH - 1
        g_wait(1, u_last)
        s_issue(1, u_last)
        s_wait(0, u_last - 1)
        s_wait(1, u_last)

    plsc.subcore_barrier()
    pltpu.sync_copy(acc.at[pl.ds(s * RPT, RPT)],
                    s_out.at[c, pl.ds(s * RPT, RPT)])


@functools.cache
def _sc_kernels():
    mesh = plsc.VectorSubcoreMesh(core_axis_name="c", subcore_axis_name="s")
    deg = pl.kernel(
        _deg_body,
        out_type=jax.ShapeDtypeStruct((NC, NP), jnp.float32),
        mesh=mesh,
        scratch_types=[
            pltpu.VMEM((NWIN, K), jnp.int32),
            pltpu.VMEM((K,), jnp.float32),
            pltpu.VMEM((RPT,), jnp.float32),
            pltpu.VMEM_SHARED((NP,), jnp.float32),
            pltpu.SemaphoreType.DMA,
        ],
    )
    agg = pl.kernel(
        _agg_body,
        out_type=jax.ShapeDtypeStruct((NC, NP, D), jnp.float32),
        mesh=mesh,
        scratch_types=[
            pltpu.VMEM((NUH, K), jnp.int32),
            pltpu.VMEM((NUH, K), jnp.int32),
            pltpu.VMEM((K, D), jnp.float32),
            pltpu.VMEM((K, D), jnp.float32),
            pltpu.VMEM_SHARED((NP, D), jnp.float32),
            pltpu.SemaphoreType.DMA,
            pltpu.SemaphoreType.DMA,
            pltpu.SemaphoreType.DMA,
            pltpu.SemaphoreType.DMA,
        ],
    )
    return deg, agg


def _prep_body(d0, d1, x, dinv_o, v0_o):
    i = pl.program_id(0)
    deg = d0[...] + d1[...] + 1.0
    dinv = lax.rsqrt(deg)
    row = i * BLK + lax.broadcasted_iota(jnp.int32, (BLK, 1), 0)
    dinv = jnp.where(row < N_NODES, dinv, 0.0)
    dinv_o[...] = dinv
    v0_o[...] = x[...] * dinv


_prep_call = pl.pallas_call(
    _prep_body,
    grid=(NP // BLK,),
    in_specs=[
        pl.BlockSpec((BLK, 1), lambda i: (i, 0)),
        pl.BlockSpec((BLK, 1), lambda i: (i, 0)),
        pl.BlockSpec((BLK, D), lambda i: (i, 0)),
    ],
    out_specs=[
        pl.BlockSpec((BLK, 1), lambda i: (i, 0)),
        pl.BlockSpec((BLK, D), lambda i: (i, 0)),
    ],
    out_shape=[
        jax.ShapeDtypeStruct((NP, 1), jnp.float32),
        jax.ShapeDtypeStruct((NP, D), jnp.float32),
    ],
)


def _layer_body(s0, s1, v, dinv, w, b, vo):
    t = (s0[...] + s1[...] + v[...]) * dinv[...]
    h = jnp.dot(t, w[...], preferred_element_type=jnp.float32) + b[...]
    vo[...] = jnp.maximum(h, 0.0) * dinv[...]


_layer_call = pl.pallas_call(
    _layer_body,
    grid=(NP // BLK,),
    in_specs=[
        pl.BlockSpec((BLK, D), lambda i: (i, 0)),
        pl.BlockSpec((BLK, D), lambda i: (i, 0)),
        pl.BlockSpec((BLK, D), lambda i: (i, 0)),
        pl.BlockSpec((BLK, 1), lambda i: (i, 0)),
        pl.BlockSpec((D, D), lambda i: (0, 0)),
        pl.BlockSpec((1, D), lambda i: (0, 0)),
    ],
    out_specs=pl.BlockSpec((BLK, D), lambda i: (i, 0)),
    out_shape=jax.ShapeDtypeStruct((NP, D), jnp.float32),
)


def _final_body(s0, s1, v, dinv, w, b, out, accs):
    i = pl.program_id(0)

    @pl.when(i == 0)
    def _():
        accs[...] = jnp.zeros_like(accs)

    t = (s0[...] + s1[...] + v[...]) * dinv[...]
    h = jnp.dot(t, w[...], preferred_element_type=jnp.float32) + b[...]
    row = i * BLK + lax.broadcasted_iota(jnp.int32, (BLK, 1), 0)
    h = jnp.where(row < N_NODES, h, 0.0)
    accs[...] += jnp.sum(h, axis=0, keepdims=True)

    @pl.when(i == pl.num_programs(0) - 1)
    def _():
        pooled = accs[...]
        lane = lax.broadcasted_iota(jnp.int32, (1, 128), 1)
        valid = lane < C
        m = jnp.max(jnp.where(valid, pooled, jnp.float32(-1e30)),
                    axis=1, keepdims=True)
        e = jnp.where(valid, jnp.exp(pooled - m), 0.0)
        ls = pooled - (m + jnp.log(jnp.sum(e, axis=1, keepdims=True)))
        out[0:1, :] = pooled
        out[1:2, :] = ls


_final_call = pl.pallas_call(
    _final_body,
    grid=(NP // BLK,),
    in_specs=[
        pl.BlockSpec((BLK, D), lambda i: (i, 0)),
        pl.BlockSpec((BLK, D), lambda i: (i, 0)),
        pl.BlockSpec((BLK, D), lambda i: (i, 0)),
        pl.BlockSpec((BLK, 1), lambda i: (i, 0)),
        pl.BlockSpec((D, D), lambda i: (0, 0)),
        pl.BlockSpec((1, D), lambda i: (0, 0)),
    ],
    out_specs=pl.BlockSpec((2, 128), lambda i: (0, 0)),
    out_shape=jax.ShapeDtypeStruct((2, 128), jnp.float32),
    scratch_shapes=[pltpu.VMEM((1, 128), jnp.float32)],
)


def kernel(x, edge_index, W1, b1, W2, b2, W3, b3, W4, b4):
    src = edge_index[0]
    dst = edge_index[1]
    pad_idx = N_NODES + (jnp.arange(EP - E, dtype=jnp.int32) % (NP - N_NODES))
    srcp = jnp.concatenate([src, pad_idx]).reshape(NC, NS, NWIN, K)
    dstp = jnp.concatenate([dst, pad_idx]).reshape(NC, NS, NWIN, K)
    xp = jnp.pad(x, ((0, NP - N_NODES), (0, 0)))

    deg_kernel, agg_kernel = _sc_kernels()
    degp = deg_kernel(dstp)
    dinv, v = _prep_call(degp[0].reshape(NP, 1), degp[1].reshape(NP, 1), xp)

    for (W, b) in ((W1, b1), (W2, b2), (W3, b3)):
        sp = agg_kernel(v, srcp, dstp)
        v = _layer_call(sp[0], sp[1], v, dinv, W, b.reshape(1, D))

    sp = agg_kernel(v, srcp, dstp)
    W4p = jnp.pad(W4, ((0, 0), (0, 128 - C)))
    b4p = jnp.pad(b4, (0, 128 - C)).reshape(1, 128)
    out = _final_call(sp[0], sp[1], v, dinv, W4p, b4p)
    return (out[0:1, :C], out[1:2, :C])


# speedup vs baseline: 22.0017x; 1.0207x over previous
import functools

import jax
import jax.numpy as jnp
from jax import lax
from jax.experimental import pallas as pl
from jax.experimental.pallas import tpu as pltpu
from jax.experimental.pallas import tpu_sc as plsc

N_NODES = 10000
D = 128
C = 40
NP = 10240
E = 320000
NC, NS = 2, 16
NWORK = NC * NS
K = 128
EP = 327680
NWIN = EP // (NWORK * K)
RPT = NP // NS
BLK = 1024


def _deg_body(dst_hbm, deg_out, didx_all, ones_v, zslab_v, acc, ssem):
    c = lax.axis_index("c")
    s = lax.axis_index("s")

    def initz(i, _):
        zslab_v[pl.ds(16 * i, 16)] = jnp.zeros((16,), jnp.float32)
        return 0

    lax.fori_loop(0, RPT // 16, initz, 0)

    def init1(i, _):
        ones_v[pl.ds(16 * i, 16)] = jnp.ones((16,), jnp.float32)
        return 0

    lax.fori_loop(0, K // 16, init1, 0)

    pltpu.sync_copy(dst_hbm.at[c, s], didx_all)
    pltpu.sync_copy(zslab_v, acc.at[pl.ds(s * RPT, RPT)])
    plsc.subcore_barrier()

    def fire(w, _):
        pltpu.async_copy(ones_v, acc.at[didx_all.at[w]], ssem, add=True)
        return 0

    lax.fori_loop(0, NWIN, fire, 0)

    def drain(w, _):
        pltpu.make_async_copy(ones_v, acc.at[didx_all.at[w]], ssem).wait()
        return 0

    lax.fori_loop(0, NWIN, drain, 0)
    plsc.subcore_barrier()
    pltpu.sync_copy(acc.at[pl.ds(s * RPT, RPT)],
                    deg_out.at[c, pl.ds(s * RPT, RPT)])


NH = 2
NUH = NWIN // NH
PROBE_NO_SCATTER = True


def _agg_body(v_hbm, src_hbm, dst_hbm, s_out, sidx_h, didx_h,
              rows0, rows1, acc, gsem0, gsem1, ssem0, ssem1):
    c = lax.axis_index("c")
    s = lax.axis_index("s")
    bufs = (rows0, rows1)
    gsems = (gsem0, gsem1)
    ssems = (ssem0, ssem1)

    def zrows(i, _):
        for j in range(D // 16):
            rows0[i, pl.ds(16 * j, 16)] = jnp.zeros((16,), jnp.float32)
        return 0

    lax.fori_loop(0, K, zrows, 0)
    base = s * RPT
    for k in range(RPT // K):
        pltpu.sync_copy(rows0, acc.at[pl.ds(base + k * K, K)])
    plsc.subcore_barrier()

    def g_issue(p, u):
        pltpu.async_copy(v_hbm.at[sidx_h.at[u]], bufs[p], gsems[p])

    def g_wait(p, u):
        pltpu.make_async_copy(v_hbm.at[sidx_h.at[u]], bufs[p],
                              gsems[p]).wait()

    def s_issue(p, u):
        if not PROBE_NO_SCATTER:
            pltpu.async_copy(bufs[p], acc.at[didx_h.at[u]], ssems[p],
                             add=True)

    def s_wait(p, u):
        if not PROBE_NO_SCATTER:
            pltpu.make_async_copy(bufs[p], acc.at[didx_h.at[u]],
                                  ssems[p]).wait()

    for h in range(NH):
        pltpu.sync_copy(src_hbm.at[c, s, pl.ds(h * NUH, NUH)], sidx_h)
        pltpu.sync_copy(dst_hbm.at[c, s, pl.ds(h * NUH, NUH)], didx_h)

        g_issue(0, 0)
        g_wait(0, 0)
        s_issue(0, 0)
        g_issue(1, 1)

        def step(k, _):
            ua = 2 * k + 1
            g_wait(1, ua)
            s_issue(1, ua)
            s_wait(0, ua - 1)
            g_issue(0, ua + 1)
            ub = 2 * k + 2
            g_wait(0, ub)
            s_issue(0, ub)
            s_wait(1, ub - 1)
            g_issue(1, ub + 1)
            return 0

        lax.fori_loop(0, (NUH - 2) // 2, step, 0)

        u_last = NUH - 1
        g_wait(1, u_last)
        s_issue(1, u_last)
        s_wait(0, u_last - 1)
        s_wait(1, u_last)

    plsc.subcore_barrier()
    pltpu.sync_copy(acc.at[pl.ds(s * RPT, RPT)],
                    s_out.at[c, pl.ds(s * RPT, RPT)])


@functools.cache
def _sc_kernels():
    mesh = plsc.VectorSubcoreMesh(core_axis_name="c", subcore_axis_name="s")
    deg = pl.kernel(
        _deg_body,
        out_type=jax.ShapeDtypeStruct((NC, NP), jnp.float32),
        mesh=mesh,
        scratch_types=[
            pltpu.VMEM((NWIN, K), jnp.int32),
            pltpu.VMEM((K,), jnp.float32),
            pltpu.VMEM((RPT,), jnp.float32),
            pltpu.VMEM_SHARED((NP,), jnp.float32),
            pltpu.SemaphoreType.DMA,
        ],
    )
    agg = pl.kernel(
        _agg_body,
        out_type=jax.ShapeDtypeStruct((NC, NP, D), jnp.float32),
        mesh=mesh,
        scratch_types=(
            [pltpu.VMEM((NUH, K), jnp.int32)] * 2
            + [pltpu.VMEM((K, D), jnp.float32)] * 2
            + [pltpu.VMEM_SHARED((NP, D), jnp.float32)]
            + [pltpu.SemaphoreType.DMA] * 4
        ),
    )
    return deg, agg


def _prep_body(d0, d1, x, dinv_o, v0_o):
    i = pl.program_id(0)
    deg = d0[...] + d1[...] + 1.0
    dinv = lax.rsqrt(deg)
    row = i * BLK + lax.broadcasted_iota(jnp.int32, (BLK, 1), 0)
    dinv = jnp.where(row < N_NODES, dinv, 0.0)
    dinv_o[...] = dinv
    v0_o[...] = x[...] * dinv


_prep_call = pl.pallas_call(
    _prep_body,
    grid=(NP // BLK,),
    in_specs=[
        pl.BlockSpec((BLK, 1), lambda i: (i, 0)),
        pl.BlockSpec((BLK, 1), lambda i: (i, 0)),
        pl.BlockSpec((BLK, D), lambda i: (i, 0)),
    ],
    out_specs=[
        pl.BlockSpec((BLK, 1), lambda i: (i, 0)),
        pl.BlockSpec((BLK, D), lambda i: (i, 0)),
    ],
    out_shape=[
        jax.ShapeDtypeStruct((NP, 1), jnp.float32),
        jax.ShapeDtypeStruct((NP, D), jnp.float32),
    ],
)


def _layer_body(s0, s1, v, dinv, w, b, vo):
    t = (s0[...] + s1[...] + v[...]) * dinv[...]
    h = jnp.dot(t, w[...], preferred_element_type=jnp.float32) + b[...]
    vo[...] = jnp.maximum(h, 0.0) * dinv[...]


_layer_call = pl.pallas_call(
    _layer_body,
    grid=(NP // BLK,),
    in_specs=[
        pl.BlockSpec((BLK, D), lambda i: (i, 0)),
        pl.BlockSpec((BLK, D), lambda i: (i, 0)),
        pl.BlockSpec((BLK, D), lambda i: (i, 0)),
        pl.BlockSpec((BLK, 1), lambda i: (i, 0)),
        pl.BlockSpec((D, D), lambda i: (0, 0)),
        pl.BlockSpec((1, D), lambda i: (0, 0)),
    ],
    out_specs=pl.BlockSpec((BLK, D), lambda i: (i, 0)),
    out_shape=jax.ShapeDtypeStruct((NP, D), jnp.float32),
)


def _final_body(s0, s1, v, dinv, w, b, out, accs):
    i = pl.program_id(0)

    @pl.when(i == 0)
    def _():
        accs[...] = jnp.zeros_like(accs)

    t = (s0[...] + s1[...] + v[...]) * dinv[...]
    h = jnp.dot(t, w[...], preferred_element_type=jnp.float32) + b[...]
    row = i * BLK + lax.broadcasted_iota(jnp.int32, (BLK, 1), 0)
    h = jnp.where(row < N_NODES, h, 0.0)
    accs[...] += jnp.sum(h, axis=0, keepdims=True)

    @pl.when(i == pl.num_programs(0) - 1)
    def _():
        pooled = accs[...]
        lane = lax.broadcasted_iota(jnp.int32, (1, 128), 1)
        valid = lane < C
        m = jnp.max(jnp.where(valid, pooled, jnp.float32(-1e30)),
                    axis=1, keepdims=True)
        e = jnp.where(valid, jnp.exp(pooled - m), 0.0)
        ls = pooled - (m + jnp.log(jnp.sum(e, axis=1, keepdims=True)))
        out[0:1, :] = pooled
        out[1:2, :] = ls


_final_call = pl.pallas_call(
    _final_body,
    grid=(NP // BLK,),
    in_specs=[
        pl.BlockSpec((BLK, D), lambda i: (i, 0)),
        pl.BlockSpec((BLK, D), lambda i: (i, 0)),
        pl.BlockSpec((BLK, D), lambda i: (i, 0)),
        pl.BlockSpec((BLK, 1), lambda i: (i, 0)),
        pl.BlockSpec((D, D), lambda i: (0, 0)),
        pl.BlockSpec((1, D), lambda i: (0, 0)),
    ],
    out_specs=pl.BlockSpec((2, 128), lambda i: (0, 0)),
    out_shape=jax.ShapeDtypeStruct((2, 128), jnp.float32),
    scratch_shapes=[pltpu.VMEM((1, 128), jnp.float32)],
)


def kernel(x, edge_index, W1, b1, W2, b2, W3, b3, W4, b4):
    src = edge_index[0]
    dst = edge_index[1]
    pad_idx = N_NODES + (jnp.arange(EP - E, dtype=jnp.int32) % (NP - N_NODES))
    srcp = jnp.concatenate([src, pad_idx]).reshape(NC, NS, NWIN, K)
    dstp = jnp.concatenate([dst, pad_idx]).reshape(NC, NS, NWIN, K)
    xp = jnp.pad(x, ((0, NP - N_NODES), (0, 0)))

    deg_kernel, agg_kernel = _sc_kernels()
    degp = deg_kernel(dstp)
    dinv, v = _prep_call(degp[0].reshape(NP, 1), degp[1].reshape(NP, 1), xp)

    for (W, b) in ((W1, b1), (W2, b2), (W3, b3)):
        sp = agg_kernel(v, srcp, dstp)
        v = _layer_call(sp[0], sp[1], v, dinv, W, b.reshape(1, D))

    sp = agg_kernel(v, srcp, dstp)
    W4p = jnp.pad(W4, ((0, 0), (0, 128 - C)))
    b4p = jnp.pad(b4, (0, 128 - C)).reshape(1, 128)
    out = _final_call(sp[0], sp[1], v, dinv, W4p, b4p)
    return (out[0:1, :C], out[1:2, :C])


# speedup vs baseline: 22.2707x; 1.0122x over previous
import functools

import jax
import jax.numpy as jnp
from jax import lax
from jax.experimental import pallas as pl
from jax.experimental.pallas import tpu as pltpu
from jax.experimental.pallas import tpu_sc as plsc

N_NODES = 10000
D = 128
C = 40
NP = 10240
E = 320000
NC, NS = 2, 16
NWORK = NC * NS
K = 128
EP = 327680
NWIN = EP // (NWORK * K)
RPT = NP // NS
BLK = 1024


def _deg_body(dst_hbm, deg_out, didx_all, ones_v, zslab_v, acc, ssem):
    c = lax.axis_index("c")
    s = lax.axis_index("s")

    def initz(i, _):
        zslab_v[pl.ds(16 * i, 16)] = jnp.zeros((16,), jnp.float32)
        return 0

    lax.fori_loop(0, RPT // 16, initz, 0)

    def init1(i, _):
        ones_v[pl.ds(16 * i, 16)] = jnp.ones((16,), jnp.float32)
        return 0

    lax.fori_loop(0, K // 16, init1, 0)

    pltpu.sync_copy(dst_hbm.at[c, s], didx_all)
    pltpu.sync_copy(zslab_v, acc.at[pl.ds(s * RPT, RPT)])
    plsc.subcore_barrier()

    def fire(w, _):
        pltpu.async_copy(ones_v, acc.at[didx_all.at[w]], ssem, add=True)
        return 0

    lax.fori_loop(0, NWIN, fire, 0)

    def drain(w, _):
        pltpu.make_async_copy(ones_v, acc.at[didx_all.at[w]], ssem).wait()
        return 0

    lax.fori_loop(0, NWIN, drain, 0)
    plsc.subcore_barrier()
    pltpu.sync_copy(acc.at[pl.ds(s * RPT, RPT)],
                    deg_out.at[c, pl.ds(s * RPT, RPT)])


NH = 2
NUH = NWIN // NH


def _agg_body(v_hbm, src_hbm, dst_hbm, s_out, sidx_h, didx_h,
              rows0, rows1, acc, gsem0, gsem1, ssem0, ssem1):
    c = lax.axis_index("c")
    s = lax.axis_index("s")
    bufs = (rows0, rows1)
    gsems = (gsem0, gsem1)
    ssems = (ssem0, ssem1)

    base = s * RPT

    @pl.when(c == 0)
    def _():
        pltpu.sync_copy(v_hbm.at[pl.ds(base, RPT)], acc.at[pl.ds(base, RPT)])

    @pl.when(c != 0)
    def _():
        def zrows(i, _):
            for j in range(D // 16):
                rows0[i, pl.ds(16 * j, 16)] = jnp.zeros((16,), jnp.float32)
            return 0

        lax.fori_loop(0, K, zrows, 0)
        for k in range(RPT // K):
            pltpu.sync_copy(rows0, acc.at[pl.ds(base + k * K, K)])

    plsc.subcore_barrier()

    def g_issue(p, u):
        pltpu.async_copy(v_hbm.at[sidx_h.at[u]], bufs[p], gsems[p])

    def g_wait(p, u):
        pltpu.make_async_copy(v_hbm.at[sidx_h.at[u]], bufs[p],
                              gsems[p]).wait()

    def s_issue(p, u):
        pltpu.async_copy(bufs[p], acc.at[didx_h.at[u]], ssems[p], add=True)

    def s_wait(p, u):
        pltpu.make_async_copy(bufs[p], acc.at[didx_h.at[u]], ssems[p]).wait()

    for h in range(NH):
        pltpu.sync_copy(src_hbm.at[c, s, pl.ds(h * NUH, NUH)], sidx_h)
        pltpu.sync_copy(dst_hbm.at[c, s, pl.ds(h * NUH, NUH)], didx_h)

        g_issue(0, 0)
        g_wait(0, 0)
        s_issue(0, 0)
        g_issue(1, 1)

        def step(k, _):
            ua = 2 * k + 1
            g_wait(1, ua)
            s_issue(1, ua)
            s_wait(0, ua - 1)
            g_issue(0, ua + 1)
            ub = 2 * k + 2
            g_wait(0, ub)
            s_issue(0, ub)
            s_wait(1, ub - 1)
            g_issue(1, ub + 1)
            return 0

        lax.fori_loop(0, (NUH - 2) // 2, step, 0)

        u_last = NUH - 1
        g_wait(1, u_last)
        s_issue(1, u_last)
        s_wait(0, u_last - 1)
        s_wait(1, u_last)

    plsc.subcore_barrier()
    pltpu.sync_copy(acc.at[pl.ds(s * RPT, RPT)],
                    s_out.at[c, pl.ds(s * RPT, RPT)])


@functools.cache
def _sc_kernels():
    mesh = plsc.VectorSubcoreMesh(core_axis_name="c", subcore_axis_name="s")
    deg = pl.kernel(
        _deg_body,
        out_type=jax.ShapeDtypeStruct((NC, NP), jnp.float32),
        mesh=mesh,
        scratch_types=[
            pltpu.VMEM((NWIN, K), jnp.int32),
            pltpu.VMEM((K,), jnp.float32),
            pltpu.VMEM((RPT,), jnp.float32),
            pltpu.VMEM_SHARED((NP,), jnp.float32),
            pltpu.SemaphoreType.DMA,
        ],
    )
    agg = pl.kernel(
        _agg_body,
        out_type=jax.ShapeDtypeStruct((NC, NP, D), jnp.float32),
        mesh=mesh,
        scratch_types=(
            [pltpu.VMEM((NUH, K), jnp.int32)] * 2
            + [pltpu.VMEM((K, D), jnp.float32)] * 2
            + [pltpu.VMEM_SHARED((NP, D), jnp.float32)]
            + [pltpu.SemaphoreType.DMA] * 4
        ),
    )
    return deg, agg


def _prep_body(d0, d1, x, dinv_o, v0_o):
    i = pl.program_id(0)
    deg = d0[...] + d1[...] + 1.0
    dinv = lax.rsqrt(deg)
    row = i * BLK + lax.broadcasted_iota(jnp.int32, (BLK, 1), 0)
    dinv = jnp.where(row < N_NODES, dinv, 0.0)
    dinv_o[...] = dinv
    v0_o[...] = x[...] * dinv


_prep_call = pl.pallas_call(
    _prep_body,
    grid=(NP // BLK,),
    in_specs=[
        pl.BlockSpec((BLK, 1), lambda i: (i, 0)),
        pl.BlockSpec((BLK, 1), lambda i: (i, 0)),
        pl.BlockSpec((BLK, D), lambda i: (i, 0)),
    ],
    out_specs=[
        pl.BlockSpec((BLK, 1), lambda i: (i, 0)),
        pl.BlockSpec((BLK, D), lambda i: (i, 0)),
    ],
    out_shape=[
        jax.ShapeDtypeStruct((NP, 1), jnp.float32),
        jax.ShapeDtypeStruct((NP, D), jnp.float32),
    ],
)


def _layer_body(sp, dinv, w, b, vo):
    t = (sp[0] + sp[1]) * dinv[...]
    h = jnp.dot(t, w[...], preferred_element_type=jnp.float32) + b[...]
    vo[...] = jnp.maximum(h, 0.0) * dinv[...]


_layer_call = pl.pallas_call(
    _layer_body,
    grid=(NP // BLK,),
    in_specs=[
        pl.BlockSpec((2, BLK, D), lambda i: (0, i, 0)),
        pl.BlockSpec((BLK, 1), lambda i: (i, 0)),
        pl.BlockSpec((D, D), lambda i: (0, 0)),
        pl.BlockSpec((1, D), lambda i: (0, 0)),
    ],
    out_specs=pl.BlockSpec((BLK, D), lambda i: (i, 0)),
    out_shape=jax.ShapeDtypeStruct((NP, D), jnp.float32),
)


def _final_body(sp, dinv, w, b, out, accs):
    i = pl.program_id(0)

    @pl.when(i == 0)
    def _():
        accs[...] = jnp.zeros_like(accs)

    t = (sp[0] + sp[1]) * dinv[...]
    h = jnp.dot(t, w[...], preferred_element_type=jnp.float32) + b[...]
    row = i * BLK + lax.broadcasted_iota(jnp.int32, (BLK, 1), 0)
    h = jnp.where(row < N_NODES, h, 0.0)
    accs[...] += jnp.sum(h, axis=0, keepdims=True)

    @pl.when(i == pl.num_programs(0) - 1)
    def _():
        pooled = accs[...]
        lane = lax.broadcasted_iota(jnp.int32, (1, 128), 1)
        valid = lane < C
        m = jnp.max(jnp.where(valid, pooled, jnp.float32(-1e30)),
                    axis=1, keepdims=True)
        e = jnp.where(valid, jnp.exp(pooled - m), 0.0)
        ls = pooled - (m + jnp.log(jnp.sum(e, axis=1, keepdims=True)))
        out[0:1, :] = pooled
        out[1:2, :] = ls


_final_call = pl.pallas_call(
    _final_body,
    grid=(NP // BLK,),
    in_specs=[
        pl.BlockSpec((2, BLK, D), lambda i: (0, i, 0)),
        pl.BlockSpec((BLK, 1), lambda i: (i, 0)),
        pl.BlockSpec((D, D), lambda i: (0, 0)),
        pl.BlockSpec((1, D), lambda i: (0, 0)),
    ],
    out_specs=pl.BlockSpec((2, 128), lambda i: (0, 0)),
    out_shape=jax.ShapeDtypeStruct((2, 128), jnp.float32),
    scratch_shapes=[pltpu.VMEM((1, 128), jnp.float32)],
)


def kernel(x, edge_index, W1, b1, W2, b2, W3, b3, W4, b4):
    src = edge_index[0]
    dst = edge_index[1]
    pad_idx = N_NODES + (jnp.arange(EP - E, dtype=jnp.int32) % (NP - N_NODES))
    srcp = jnp.concatenate([src, pad_idx]).reshape(NC, NS, NWIN, K)
    dstp = jnp.concatenate([dst, pad_idx]).reshape(NC, NS, NWIN, K)
    xp = jnp.pad(x, ((0, NP - N_NODES), (0, 0)))

    deg_kernel, agg_kernel = _sc_kernels()
    degp = deg_kernel(dstp)
    dinv, v = _prep_call(degp[0].reshape(NP, 1), degp[1].reshape(NP, 1), xp)

    for (W, b) in ((W1, b1), (W2, b2), (W3, b3)):
        sp = agg_kernel(v, srcp, dstp)
        v = _layer_call(sp, dinv, W, b.reshape(1, D))

    sp = agg_kernel(v, srcp, dstp)
    W4p = jnp.pad(W4, ((0, 0), (0, 128 - C)))
    b4p = jnp.pad(b4, (0, 128 - C)).reshape(1, 128)
    out = _final_call(sp, dinv, W4p, b4p)
    return (out[0:1, :C], out[1:2, :C])
